# baseline (speedup 1.0000x reference)
import functools

import jax
import jax.numpy as jnp
from jax import lax
from jax.experimental import pallas as pl
from jax.experimental.pallas import tpu as pltpu
from jax.experimental.pallas import tpu_sc as plsc

VOCAB = 30522
HIDDEN = 768
MAX_POS = 512
BATCH = 32
SEQ = 512
EPS = 1e-12

NC = 2
NS = 16
NW = NC * NS

NCH = 4
B_CH = BATCH // NCH
T_CH = B_CH * SEQ
B_PER_W = T_CH // NW
G = 64
NCHUNK = B_PER_W // G


def _sc_gather(word_emb, idx3):
    mesh = plsc.VectorSubcoreMesh(core_axis_name="c", subcore_axis_name="s")

    @functools.partial(
        pl.kernel,
        mesh=mesh,
        out_type=jax.ShapeDtypeStruct((T_CH, HIDDEN), jnp.float32),
        scratch_types=[
            pltpu.VMEM((NCHUNK, G), jnp.int32),
            pltpu.VMEM((G, HIDDEN), jnp.float32),
            pltpu.VMEM((G, HIDDEN), jnp.float32),
            pltpu.SemaphoreType.DMA,
            pltpu.SemaphoreType.DMA,
        ],
    )
    def k(table_hbm, idx_hbm, out_hbm, idx_v, rows0, rows1, sem0, sem1):
        wid = lax.axis_index("s") * NC + lax.axis_index("c")
        base = wid * B_PER_W
        pltpu.sync_copy(idx_hbm.at[wid], idx_v)
        bufs = (rows0, rows1)
        sems = (sem0, sem1)
        copies = [None] * NCHUNK
        copies[0] = pltpu.async_copy(table_hbm.at[idx_v.at[0]], bufs[0], sems[0])
        for j in range(NCHUNK):
            if j + 1 < NCHUNK:
                copies[j + 1] = pltpu.async_copy(
                    table_hbm.at[idx_v.at[j + 1]], bufs[(j + 1) % 2], sems[(j + 1) % 2]
                )
            copies[j].wait()
            pltpu.sync_copy(bufs[j % 2], out_hbm.at[pl.ds(base + j * G, G)])

    return k(word_emb, idx3)


def _ln_body(acc_ref, g_ref, p_ref, gamma_ref, beta_ref, o_ref):
    del acc_ref
    x = g_ref[...] + p_ref[...]
    mu = jnp.mean(x, axis=1, keepdims=True)
    xc = x - mu
    var = jnp.mean(xc * xc, axis=1, keepdims=True)
    y = xc * lax.rsqrt(var + EPS)
    o_ref[...] = (y * gamma_ref[...] + beta_ref[...])[None]


def _tc_add_ln_chunk(acc, gathered, pos_emb, gamma, beta, chunk):
    return pl.pallas_call(
        _ln_body,
        grid=(B_CH,),
        in_specs=[
            pl.BlockSpec(memory_space=pl.ANY),
            pl.BlockSpec((SEQ, HIDDEN), lambda i: (i, 0)),
            pl.BlockSpec((SEQ, HIDDEN), lambda i: (0, 0)),
            pl.BlockSpec((1, HIDDEN), lambda i: (0, 0)),
            pl.BlockSpec((1, HIDDEN), lambda i: (0, 0)),
        ],
        out_specs=pl.BlockSpec(
            (1, SEQ, HIDDEN), lambda i, _c=chunk: (_c * B_CH + i, 0, 0)
        ),
        out_shape=jax.ShapeDtypeStruct((BATCH, SEQ, HIDDEN), jnp.float32),
        input_output_aliases={0: 0},
    )(acc, gathered, pos_emb, gamma, beta)


def kernel(input_ids, token_type_ids, word_emb, pos_emb, ln_gamma, ln_beta):
    del token_type_ids
    ids = input_ids.astype(jnp.int32).reshape(NCH, NW, NCHUNK, G)
    gamma = ln_gamma.reshape(1, HIDDEN)
    beta = ln_beta.reshape(1, HIDDEN)
    gathered = [_sc_gather(word_emb, ids[k]) for k in range(NCH)]
    acc = jnp.zeros((BATCH, SEQ, HIDDEN), jnp.float32)
    for k in range(NCH):
        acc = _tc_add_ln_chunk(acc, gathered[k], pos_emb, gamma, beta, k)
    return acc


# speedup vs baseline: 1.1553x; 1.1553x over previous
import functools

import jax
import jax.numpy as jnp
from jax import lax
from jax.experimental import pallas as pl
from jax.experimental.pallas import tpu as pltpu
from jax.experimental.pallas import tpu_sc as plsc

VOCAB = 30522
HIDDEN = 768
MAX_POS = 512
BATCH = 32
SEQ = 512
EPS = 1e-12

NC = 2
NS = 16
NW = NC * NS

NCH = 4
B_CH = BATCH // NCH
T_CH = B_CH * SEQ
B_PER_W = T_CH // NW
G = 64
NCHUNK = B_PER_W // G


def _sc_gather(word_emb, idx3):
    mesh = plsc.VectorSubcoreMesh(core_axis_name="c", subcore_axis_name="s")

    @functools.partial(
        pl.kernel,
        mesh=mesh,
        out_type=jax.ShapeDtypeStruct((T_CH, HIDDEN), jnp.float32),
        scratch_types=[
            pltpu.VMEM((NCHUNK, G), jnp.int32),
            pltpu.VMEM((G, HIDDEN), jnp.float32),
            pltpu.VMEM((G, HIDDEN), jnp.float32),
            pltpu.SemaphoreType.DMA,
            pltpu.SemaphoreType.DMA,
        ],
    )
    def k(table_hbm, idx_hbm, out_hbm, idx_v, rows0, rows1, sem0, sem1):
        wid = lax.axis_index("s") * NC + lax.axis_index("c")
        base = wid * B_PER_W
        pltpu.sync_copy(idx_hbm.at[wid], idx_v)
        bufs = (rows0, rows1)
        sems = (sem0, sem1)
        copies = [None] * NCHUNK
        copies[0] = pltpu.async_copy(table_hbm.at[idx_v.at[0]], bufs[0], sems[0])
        for j in range(NCHUNK):
            if j + 1 < NCHUNK:
                copies[j + 1] = pltpu.async_copy(
                    table_hbm.at[idx_v.at[j + 1]], bufs[(j + 1) % 2], sems[(j + 1) % 2]
                )
            copies[j].wait()
            pltpu.sync_copy(bufs[j % 2], out_hbm.at[pl.ds(base + j * G, G)])

    return k(word_emb, idx3)


def _ln_body(g_ref, p_ref, gamma_ref, beta_ref, o_ref):
    x = g_ref[...] + p_ref[...]
    mu = jnp.mean(x, axis=1, keepdims=True)
    xc = x - mu
    var = jnp.mean(xc * xc, axis=1, keepdims=True)
    y = xc * lax.rsqrt(var + EPS)
    o_ref[...] = (y * gamma_ref[...] + beta_ref[...])[None]


def _tc_add_ln_chunk(acc, gathered, pos_emb, gamma, beta, chunk):
    data_specs = [
        pl.BlockSpec((SEQ, HIDDEN), lambda i: (i, 0)),
        pl.BlockSpec((SEQ, HIDDEN), lambda i: (0, 0)),
        pl.BlockSpec((1, HIDDEN), lambda i: (0, 0)),
        pl.BlockSpec((1, HIDDEN), lambda i: (0, 0)),
    ]
    if acc is None:
        in_specs, args, aliases, body = data_specs, (), {}, _ln_body
    else:
        def body(acc_ref, *refs):
            del acc_ref
            _ln_body(*refs)

        in_specs = [pl.BlockSpec(memory_space=pl.ANY)] + data_specs
        args, aliases = (acc,), {0: 0}
    return pl.pallas_call(
        body,
        grid=(B_CH,),
        in_specs=in_specs,
        out_specs=pl.BlockSpec(
            (1, SEQ, HIDDEN), lambda i, _c=chunk: (_c * B_CH + i, 0, 0)
        ),
        out_shape=jax.ShapeDtypeStruct((BATCH, SEQ, HIDDEN), jnp.float32),
        input_output_aliases=aliases,
    )(*args, gathered, pos_emb, gamma, beta)


def kernel(input_ids, token_type_ids, word_emb, pos_emb, ln_gamma, ln_beta):
    del token_type_ids
    ids = input_ids.astype(jnp.int32).reshape(NCH, NW, NCHUNK, G)
    gamma = ln_gamma.reshape(1, HIDDEN)
    beta = ln_beta.reshape(1, HIDDEN)
    gathered = [_sc_gather(word_emb, ids[k]) for k in range(NCH)]
    acc = None
    for k in range(NCH):
        acc = _tc_add_ln_chunk(acc, gathered[k], pos_emb, gamma, beta, k)
    return acc
